# baseline (speedup 1.0000x reference)
import functools

import jax
import jax.numpy as jnp
from jax.experimental import pallas as pl
from jax.experimental.pallas import tpu as pltpu

_PL = 8
_PW = 16
_BB = 8
_BF = jnp.bfloat16
_VMEM = 64 * 1024 * 1024


def _cp():
    return pltpu.CompilerParams(dimension_semantics=("parallel",),
                                vmem_limit_bytes=_VMEM)


def _full(shape):
    shape = tuple(int(s) for s in shape)
    z = (0,) * len(shape)
    return pl.BlockSpec(shape, lambda b: z)


def _bspec(bb, shape):
    shape = tuple(int(s) for s in shape)
    z = (0,) * len(shape)
    return pl.BlockSpec((bb,) + shape, lambda b: (b,) + z)


def _conv3(pad, x, w_ref, b_ref, relu):
    N, H, W, C = x.shape
    Cout = int(w_ref.shape[-1])
    xb = x.astype(pad.dtype)
    pad[pl.ds(0, N), pl.ds(2, H), pl.ds(_PL, W), pl.ds(0, C)] = xb
    pad[pl.ds(0, N), pl.ds(1, H), pl.ds(_PL, W), pl.ds(C, C)] = xb
    pad[pl.ds(0, N), pl.ds(0, H), pl.ds(_PL, W), pl.ds(2 * C, C)] = xb
    zr = jnp.zeros((N, 1, W + 2, C), pad.dtype)
    zc = jnp.zeros((N, H, 1, 3 * C), pad.dtype)
    pad[pl.ds(0, N), pl.ds(1, 1), pl.ds(_PL - 1, W + 2), pl.ds(0, C)] = zr
    pad[pl.ds(0, N), pl.ds(H, 1), pl.ds(_PL - 1, W + 2), pl.ds(2 * C, C)] = zr
    pad[pl.ds(0, N), pl.ds(1, H), pl.ds(_PL - 1, 1), pl.ds(0, 3 * C)] = zc
    pad[pl.ds(0, N), pl.ds(1, H), pl.ds(_PL + W, 1), pl.ds(0, 3 * C)] = zc
    b = b_ref[...]
    RB = max(1, 2048 // (N * W))
    chunks = []
    for r0 in range(0, H, RB):
        rb = min(RB, H - r0)
        acc = None
        for dx in range(3):
            win = pad[pl.ds(0, N), pl.ds(1 + r0, rb),
                      pl.ds(_PL - 1 + dx, W), pl.ds(0, 3 * C)]
            d = jnp.dot(win.reshape(N * rb * W, 3 * C), w_ref[dx],
                        preferred_element_type=jnp.float32)
            acc = d if acc is None else acc + d
        acc = acc + b
        if relu:
            acc = jnp.maximum(acc, 0.0)
        acc = jnp.clip(acc, -1.0, 1.0)
        chunks.append(acc.reshape(N, rb, W, Cout))
    return chunks[0] if len(chunks) == 1 else jnp.concatenate(chunks, axis=1)


def _conv1(x, w_ref, b_ref, relu):
    N, H, W, C = x.shape
    Cout = int(w_ref.shape[-1])
    acc = jnp.dot(x.reshape(N * H * W, C).astype(_BF), w_ref[...],
                  preferred_element_type=jnp.float32) + b_ref[...]
    if relu:
        acc = jnp.maximum(acc, 0.0)
    return jnp.clip(acc, -1.0, 1.0).reshape(N, H, W, Cout)


def _pool(x):
    N, H2, W2, C = x.shape
    v = x.reshape(N, H2 // 2, 2, W2, C)
    m = jnp.maximum(v[:, :, 0], v[:, :, 1])
    v2 = m.reshape(N, H2 // 2, W2 // 2, 2, C)
    return jnp.maximum(v2[:, :, :, 0], v2[:, :, :, 1])


def _res(pad_a, pad_b, x, wp, bp, w0, b0, w1, b1, add):
    x1 = _conv3(pad_a, x, wp, bp, True)
    y = _conv3(pad_b, x1, w0, b0, True)
    y = _conv3(pad_b, y, w1, b1, True)
    if not add:
        return y
    return jnp.clip(y + x1, -1.0, 1.0)


def _up2(pad, prev, skip, wu, bu, wp, bp):
    N, Hh, Wh, C = prev.shape
    pad[pl.ds(0, N), pl.ds(1, Hh), pl.ds(_PL, Wh), pl.ds(0, C)] = prev.astype(pad.dtype)
    pad[pl.ds(0, N), pl.ds(Hh + 1, 1), pl.ds(_PL, Wh + 1), pl.ds(0, C)] = (
        jnp.zeros((N, 1, Wh + 1, C), pad.dtype))
    pad[pl.ds(0, N), pl.ds(1, Hh), pl.ds(_PL + Wh, 1), pl.ds(0, C)] = (
        jnp.zeros((N, Hh, 1, C), pad.dtype))
    xs = {}
    for sh in range(2):
        for sw in range(2):
            xs[(sh, sw)] = pad[pl.ds(0, N), pl.ds(1 + sh, Hh),
                               pl.ds(_PL + sw, Wh),
                               pl.ds(0, C)].reshape(N * Hh * Wh, C)
    phases = {(0, 0): ((0, 0, 1, 1),),
              (0, 1): ((0, 1, 1, 0), (0, 0, 1, 2)),
              (1, 0): ((1, 0, 0, 1), (0, 0, 2, 1)),
              (1, 1): ((1, 1, 0, 0), (1, 0, 0, 2), (0, 1, 2, 0), (0, 0, 2, 2))}
    b = bu[...]
    ph = {}
    for (py, px), taps in phases.items():
        acc = None
        for (sh, sw, ky, kx) in taps:
            d = jnp.dot(xs[(sh, sw)], wu[ky, kx],
                        preferred_element_type=jnp.float32)
            acc = d if acc is None else acc + d
        ph[(py, px)] = jnp.clip(acc + b, -1.0, 1.0).reshape(N, Hh, 1, Wh, 1, C)
    r0 = jnp.concatenate([ph[(0, 0)], ph[(0, 1)]], axis=4)
    r1 = jnp.concatenate([ph[(1, 0)], ph[(1, 1)]], axis=4)
    up = jnp.concatenate([r0, r1], axis=2).reshape(N, 2 * Hh, 2 * Wh, C)
    y = jnp.clip(up + skip, -1.0, 1.0)
    return _conv3(pad, y, wp, bp, True)


def _stem_body(*refs):
    (cols_ref, w1, b1, w2, b2,
     r0p, r0pb, r0c0, r0c0b, r0c1, r0c1b,
     r1p, r1pb, r1c0, r1c0b, r1c1, r1c1b,
     r2p, r2pb, r2c0, r2c0b, r2c1, r2c1b,
     o_ref, pad) = refs
    H, W = 64, 64
    cols = cols_ref[...]
    h = jnp.dot(cols.reshape(H * W, 27), w1[...],
                preferred_element_type=jnp.float32) + b1[...]
    h = jnp.clip(jnp.maximum(h, 0.0), -1.0, 1.0).reshape(1, H, W, 64)
    h = _conv3(pad, h, w2, b2, True)
    y = _res(pad, pad, _pool(h), r0p, r0pb, r0c0, r0c0b, r0c1, r0c1b, False)
    y = _res(pad, pad, _pool(y), r1p, r1pb, r1c0, r1c0b, r1c1, r1c1b, True)
    y = _res(pad, pad, y, r2p, r2pb, r2c0, r2c0b, r2c1, r2c1b, True)
    o_ref[...] = y.astype(o_ref.dtype)


@functools.lru_cache(maxsize=None)
def _get_stem_call(B):
    n_w = 22
    return lambda *a: pl.pallas_call(
        _stem_body,
        out_shape=jax.ShapeDtypeStruct((B, 16, 16, 64), _BF),
        grid=(B,),
        in_specs=[_bspec(1, (64, 64, 27))] + [_full(x.shape) for x in a[1:]],
        out_specs=_bspec(1, (16, 16, 64)),
        scratch_shapes=[pltpu.VMEM((1, 66, 64 + _PW, 192), _BF)],
        compiler_params=_cp())(*a)


def _tail_body(*refs):
    it = iter(refs)

    def nxt(n):
        return [next(it) for _ in range(n)]

    (x_ref,) = nxt(1)
    rbw = [nxt(6) for _ in range(4)]
    sk = [nxt(2) for _ in range(4)]
    fw = [nxt(2) for _ in range(6)]
    clsw = nxt(14)
    regw = nxt(14)
    oreg = nxt(4)
    ocls = nxt(4)
    p64, p128 = nxt(2)

    x = x_ref[...].astype(jnp.float32)
    e32 = _res(p64, p64, _pool(x), *rbw[0], True)
    e16 = _res(p64, p64, _pool(e32), *rbw[1], True)
    e8 = _res(p64, p128, _pool(e16), *rbw[2], True)
    e4 = _res(p128, p128, _pool(e8), *rbw[3], True)
    s32 = _conv1(e32, *sk[0], True)
    s16 = _conv1(e16, *sk[1], True)
    s8 = _conv1(e8, *sk[2], True)
    s4 = _conv1(e4, *sk[3], True)
    f4 = s4
    f8 = _up2(p64, f4, s8, *fw[0], *fw[1])
    f16 = _up2(p64, f8, s16, *fw[2], *fw[3])
    f32l = _up2(p64, f16, s32, *fw[4], *fw[5])

    def head(f, w):
        h = _res(p64, p64, f, *w[0:6], True)
        h = _res(p64, p64, h, *w[6:12], True)
        return _conv3(p64, h, w[12], w[13], False)

    for o, f in zip(oreg, (f32l, f16, f8, f4)):
        o[...] = head(f, regw)
    for o, f in zip(ocls, (f32l, f16, f8, f4)):
        o[...] = head(f, clsw)


@functools.lru_cache(maxsize=None)
def _get_tail_call(B, bb):
    shapes = ((8, 8), (4, 4), (2, 2), (1, 1))
    out_shape = tuple(jax.ShapeDtypeStruct((B, h, w, 24), jnp.float32)
                      for h, w in shapes)
    out_shape += tuple(jax.ShapeDtypeStruct((B, h, w, 126), jnp.float32)
                       for h, w in shapes)
    out_specs = tuple(_bspec(bb, (h, w, 24)) for h, w in shapes)
    out_specs += tuple(_bspec(bb, (h, w, 126)) for h, w in shapes)
    return lambda *a: pl.pallas_call(
        _tail_body,
        out_shape=out_shape,
        grid=(B // bb,),
        in_specs=[_bspec(bb, (16, 16, 64))] + [_full(x.shape) for x in a[1:]],
        out_specs=out_specs,
        scratch_shapes=[pltpu.VMEM((bb, 10, 8 + _PW, 192), _BF),
                        pltpu.VMEM((bb, 4, 2 + _PW, 384), _BF)],
        compiler_params=_cp())(*a)


def kernel(x, pre1_w, pre1_b, pre2_w, pre2_b, rb0_pre_w, rb0_pre_b, rb0_c0_w, rb0_c0_b, rb0_c1_w, rb0_c1_b, rb1_pre_w, rb1_pre_b, rb1_c0_w, rb1_c0_b, rb1_c1_w, rb1_c1_b, rb2_pre_w, rb2_pre_b, rb2_c0_w, rb2_c0_b, rb2_c1_w, rb2_c1_b, rb3_pre_w, rb3_pre_b, rb3_c0_w, rb3_c0_b, rb3_c1_w, rb3_c1_b, rb4_pre_w, rb4_pre_b, rb4_c0_w, rb4_c0_b, rb4_c1_w, rb4_c1_b, rb5_pre_w, rb5_pre_b, rb5_c0_w, rb5_c0_b, rb5_c1_w, rb5_c1_b, rb6_pre_w, rb6_pre_b, rb6_c0_w, rb6_c0_b, rb6_c1_w, rb6_c1_b, fpn_skip32_w, fpn_skip32_b, fpn_skip16_w, fpn_skip16_b, fpn_skip8_w, fpn_skip8_b, fpn_skip4_w, fpn_skip4_b, fpn_up4_w, fpn_up4_b, fpn_proc8_w, fpn_proc8_b, fpn_up8_w, fpn_up8_b, fpn_proc16_w, fpn_proc16_b, fpn_up16_w, fpn_up16_b, fpn_proc32_w, fpn_proc32_b, cls_res1_pre_w, cls_res1_pre_b, cls_res1_c0_w, cls_res1_c0_b, cls_res1_c1_w, cls_res1_c1_b, cls_res2_pre_w, cls_res2_pre_b, cls_res2_c0_w, cls_res2_c0_b, cls_res2_c1_w, cls_res2_c1_b, cls_conv5_w, cls_conv5_b, reg_res1_pre_w, reg_res1_pre_b, reg_res1_c0_w, reg_res1_c0_b, reg_res1_c1_w, reg_res1_c1_b, reg_res2_pre_w, reg_res2_pre_b, reg_res2_c0_w, reg_res2_c0_b, reg_res2_c1_w, reg_res2_c1_b, reg_conv5_w, reg_conv5_b):
    B = x.shape[0]
    bb = _BB if B % _BB == 0 else 1

    xh = jnp.transpose(x, (0, 2, 3, 1)).astype(jnp.float32)
    xp = jnp.pad(xh, ((0, 0), (1, 1), (1, 1), (0, 0)))
    cols = jnp.concatenate(
        [xp[:, dy:dy + 64, dx:dx + 64, :] for dy in range(3) for dx in range(3)],
        axis=-1).astype(_BF)

    def wb(w, b):
        return [w.astype(_BF), b.reshape(1, -1)]

    def wb3(w, b):
        c3 = w.shape[0] * w.shape[2]
        ws = w.transpose(1, 0, 2, 3).reshape(3, c3, w.shape[3]).astype(_BF)
        return [ws, b.reshape(1, -1)]

    stem_args = [cols, pre1_w.reshape(27, 64).astype(_BF), pre1_b.reshape(1, -1)]
    for w, b in ((pre2_w, pre2_b),
                 (rb0_pre_w, rb0_pre_b), (rb0_c0_w, rb0_c0_b), (rb0_c1_w, rb0_c1_b),
                 (rb1_pre_w, rb1_pre_b), (rb1_c0_w, rb1_c0_b), (rb1_c1_w, rb1_c1_b),
                 (rb2_pre_w, rb2_pre_b), (rb2_c0_w, rb2_c0_b), (rb2_c1_w, rb2_c1_b)):
        stem_args += wb3(w, b)
    enc = _get_stem_call(B)(*stem_args)

    tail_args = [enc]
    for kind, w, b in (
            (3, rb3_pre_w, rb3_pre_b), (3, rb3_c0_w, rb3_c0_b), (3, rb3_c1_w, rb3_c1_b),
            (3, rb4_pre_w, rb4_pre_b), (3, rb4_c0_w, rb4_c0_b), (3, rb4_c1_w, rb4_c1_b),
            (3, rb5_pre_w, rb5_pre_b), (3, rb5_c0_w, rb5_c0_b), (3, rb5_c1_w, rb5_c1_b),
            (3, rb6_pre_w, rb6_pre_b), (3, rb6_c0_w, rb6_c0_b), (3, rb6_c1_w, rb6_c1_b),
            (1, fpn_skip32_w, fpn_skip32_b), (1, fpn_skip16_w, fpn_skip16_b),
            (1, fpn_skip8_w, fpn_skip8_b), (1, fpn_skip4_w, fpn_skip4_b),
            (1, fpn_up4_w, fpn_up4_b), (3, fpn_proc8_w, fpn_proc8_b),
            (1, fpn_up8_w, fpn_up8_b), (3, fpn_proc16_w, fpn_proc16_b),
            (1, fpn_up16_w, fpn_up16_b), (3, fpn_proc32_w, fpn_proc32_b),
            (3, cls_res1_pre_w, cls_res1_pre_b), (3, cls_res1_c0_w, cls_res1_c0_b),
            (3, cls_res1_c1_w, cls_res1_c1_b),
            (3, cls_res2_pre_w, cls_res2_pre_b), (3, cls_res2_c0_w, cls_res2_c0_b),
            (3, cls_res2_c1_w, cls_res2_c1_b),
            (3, cls_conv5_w, cls_conv5_b),
            (3, reg_res1_pre_w, reg_res1_pre_b), (3, reg_res1_c0_w, reg_res1_c0_b),
            (3, reg_res1_c1_w, reg_res1_c1_b),
            (3, reg_res2_pre_w, reg_res2_pre_b), (3, reg_res2_c0_w, reg_res2_c0_b),
            (3, reg_res2_c1_w, reg_res2_c1_b),
            (3, reg_conv5_w, reg_conv5_b)):
        tail_args += wb3(w, b) if kind == 3 else wb(w, b)
    outs = _get_tail_call(B, bb)(*tail_args)
    reg_outs, cls_outs = outs[:4], outs[4:]

    def flat(o, k):
        Bo, H, W, C = o.shape
        return o.reshape(Bo, H * W * (C // k), k)

    regression = jnp.concatenate([flat(o, 4) for o in reg_outs], axis=1)
    classification = jnp.concatenate([flat(o, 21) for o in cls_outs], axis=1)
    return regression, classification


# speedup vs baseline: 1.0551x; 1.0551x over previous
import functools

import jax
import jax.numpy as jnp
from jax.experimental import pallas as pl
from jax.experimental.pallas import tpu as pltpu

_PL = 8
_PW = 16
_BB = 16
_BF = jnp.bfloat16
_VMEM = 64 * 1024 * 1024


def _cp():
    return pltpu.CompilerParams(dimension_semantics=("parallel",),
                                vmem_limit_bytes=_VMEM)


def _full(shape):
    shape = tuple(int(s) for s in shape)
    z = (0,) * len(shape)
    return pl.BlockSpec(shape, lambda b: z)


def _bspec(bb, shape):
    shape = tuple(int(s) for s in shape)
    z = (0,) * len(shape)
    return pl.BlockSpec((bb,) + shape, lambda b: (b,) + z)


def _conv3(pad, x, w_ref, b_ref, relu):
    N, H, W, C = x.shape
    Cout = int(w_ref.shape[-1])
    xb = x.astype(pad.dtype)
    pad[pl.ds(0, N), pl.ds(2, H), pl.ds(_PL, W), pl.ds(0, C)] = xb
    pad[pl.ds(0, N), pl.ds(1, H), pl.ds(_PL, W), pl.ds(C, C)] = xb
    pad[pl.ds(0, N), pl.ds(0, H), pl.ds(_PL, W), pl.ds(2 * C, C)] = xb
    zr = jnp.zeros((N, 1, W + 2, C), pad.dtype)
    zc = jnp.zeros((N, H, 1, 3 * C), pad.dtype)
    pad[pl.ds(0, N), pl.ds(1, 1), pl.ds(_PL - 1, W + 2), pl.ds(0, C)] = zr
    pad[pl.ds(0, N), pl.ds(H, 1), pl.ds(_PL - 1, W + 2), pl.ds(2 * C, C)] = zr
    pad[pl.ds(0, N), pl.ds(1, H), pl.ds(_PL - 1, 1), pl.ds(0, 3 * C)] = zc
    pad[pl.ds(0, N), pl.ds(1, H), pl.ds(_PL + W, 1), pl.ds(0, 3 * C)] = zc
    b = b_ref[...]
    RB = max(1, 2048 // (N * W))
    chunks = []
    for r0 in range(0, H, RB):
        rb = min(RB, H - r0)
        acc = None
        for dx in range(3):
            win = pad[pl.ds(0, N), pl.ds(1 + r0, rb),
                      pl.ds(_PL - 1 + dx, W), pl.ds(0, 3 * C)]
            d = jnp.dot(win.reshape(N * rb * W, 3 * C), w_ref[dx],
                        preferred_element_type=jnp.float32)
            acc = d if acc is None else acc + d
        acc = acc + b
        if relu:
            acc = jnp.maximum(acc, 0.0)
        acc = jnp.clip(acc, -1.0, 1.0)
        chunks.append(acc.reshape(N, rb, W, Cout))
    return chunks[0] if len(chunks) == 1 else jnp.concatenate(chunks, axis=1)


def _conv1(x, w_ref, b_ref, relu):
    N, H, W, C = x.shape
    Cout = int(w_ref.shape[-1])
    acc = jnp.dot(x.reshape(N * H * W, C).astype(_BF), w_ref[...],
                  preferred_element_type=jnp.float32) + b_ref[...]
    if relu:
        acc = jnp.maximum(acc, 0.0)
    return jnp.clip(acc, -1.0, 1.0).reshape(N, H, W, Cout)


def _pool(x):
    N, H2, W2, C = x.shape
    v = x.reshape(N, H2 // 2, 2, W2, C)
    m = jnp.maximum(v[:, :, 0], v[:, :, 1])
    v2 = m.reshape(N, H2 // 2, W2 // 2, 2, C)
    return jnp.maximum(v2[:, :, :, 0], v2[:, :, :, 1])


def _res(pad_a, pad_b, x, wp, bp, w0, b0, w1, b1, add):
    x1 = _conv3(pad_a, x, wp, bp, True)
    y = _conv3(pad_b, x1, w0, b0, True)
    y = _conv3(pad_b, y, w1, b1, True)
    if not add:
        return y
    return jnp.clip(y + x1, -1.0, 1.0)


def _up2(pad, prev, skip, wu, bu, wp, bp):
    N, Hh, Wh, C = prev.shape
    pad[pl.ds(0, N), pl.ds(1, Hh), pl.ds(_PL, Wh), pl.ds(0, C)] = prev.astype(pad.dtype)
    pad[pl.ds(0, N), pl.ds(Hh + 1, 1), pl.ds(_PL, Wh + 1), pl.ds(0, C)] = (
        jnp.zeros((N, 1, Wh + 1, C), pad.dtype))
    pad[pl.ds(0, N), pl.ds(1, Hh), pl.ds(_PL + Wh, 1), pl.ds(0, C)] = (
        jnp.zeros((N, Hh, 1, C), pad.dtype))
    xs = {}
    for sh in range(2):
        for sw in range(2):
            xs[(sh, sw)] = pad[pl.ds(0, N), pl.ds(1 + sh, Hh),
                               pl.ds(_PL + sw, Wh),
                               pl.ds(0, C)].reshape(N * Hh * Wh, C)
    phases = {(0, 0): ((0, 0, 1, 1),),
              (0, 1): ((0, 1, 1, 0), (0, 0, 1, 2)),
              (1, 0): ((1, 0, 0, 1), (0, 0, 2, 1)),
              (1, 1): ((1, 1, 0, 0), (1, 0, 0, 2), (0, 1, 2, 0), (0, 0, 2, 2))}
    b = bu[...]
    ph = {}
    for (py, px), taps in phases.items():
        acc = None
        for (sh, sw, ky, kx) in taps:
            d = jnp.dot(xs[(sh, sw)], wu[ky, kx],
                        preferred_element_type=jnp.float32)
            acc = d if acc is None else acc + d
        ph[(py, px)] = jnp.clip(acc + b, -1.0, 1.0).reshape(N, Hh, 1, Wh, 1, C)
    r0 = jnp.concatenate([ph[(0, 0)], ph[(0, 1)]], axis=4)
    r1 = jnp.concatenate([ph[(1, 0)], ph[(1, 1)]], axis=4)
    up = jnp.concatenate([r0, r1], axis=2).reshape(N, 2 * Hh, 2 * Wh, C)
    y = jnp.clip(up + skip, -1.0, 1.0)
    return _conv3(pad, y, wp, bp, True)


def _stem_body(*refs):
    (cols_ref, w1, b1, w2, b2,
     r0p, r0pb, r0c0, r0c0b, r0c1, r0c1b,
     r1p, r1pb, r1c0, r1c0b, r1c1, r1c1b,
     r2p, r2pb, r2c0, r2c0b, r2c1, r2c1b,
     o_ref, pad) = refs
    H, W = 64, 64
    cols = cols_ref[...]
    h = jnp.dot(cols.reshape(H * W, 27), w1[...],
                preferred_element_type=jnp.float32) + b1[...]
    h = jnp.clip(jnp.maximum(h, 0.0), -1.0, 1.0).reshape(1, H, W, 64)
    h = _conv3(pad, h, w2, b2, True)
    y = _res(pad, pad, _pool(h), r0p, r0pb, r0c0, r0c0b, r0c1, r0c1b, False)
    y = _res(pad, pad, _pool(y), r1p, r1pb, r1c0, r1c0b, r1c1, r1c1b, True)
    y = _res(pad, pad, y, r2p, r2pb, r2c0, r2c0b, r2c1, r2c1b, True)
    o_ref[...] = y.astype(o_ref.dtype)


@functools.lru_cache(maxsize=None)
def _get_stem_call(B):
    n_w = 22
    return lambda *a: pl.pallas_call(
        _stem_body,
        out_shape=jax.ShapeDtypeStruct((B, 16, 16, 64), _BF),
        grid=(B,),
        in_specs=[_bspec(1, (64, 64, 27))] + [_full(x.shape) for x in a[1:]],
        out_specs=_bspec(1, (16, 16, 64)),
        scratch_shapes=[pltpu.VMEM((1, 66, 64 + _PW, 192), _BF)],
        compiler_params=_cp())(*a)


def _tail_body(*refs):
    it = iter(refs)

    def nxt(n):
        return [next(it) for _ in range(n)]

    (x_ref,) = nxt(1)
    rbw = [nxt(6) for _ in range(4)]
    sk = [nxt(2) for _ in range(4)]
    fw = [nxt(2) for _ in range(6)]
    clsw = nxt(14)
    regw = nxt(14)
    oreg = nxt(4)
    ocls = nxt(4)
    p64, p128 = nxt(2)

    x = x_ref[...].astype(jnp.float32)
    e32 = _res(p64, p64, _pool(x), *rbw[0], True)
    e16 = _res(p64, p64, _pool(e32), *rbw[1], True)
    e8 = _res(p64, p128, _pool(e16), *rbw[2], True)
    e4 = _res(p128, p128, _pool(e8), *rbw[3], True)
    s32 = _conv1(e32, *sk[0], True)
    s16 = _conv1(e16, *sk[1], True)
    s8 = _conv1(e8, *sk[2], True)
    s4 = _conv1(e4, *sk[3], True)
    f4 = s4
    f8 = _up2(p64, f4, s8, *fw[0], *fw[1])
    f16 = _up2(p64, f8, s16, *fw[2], *fw[3])
    f32l = _up2(p64, f16, s32, *fw[4], *fw[5])

    def head(f, w):
        h = _res(p64, p64, f, *w[0:6], True)
        h = _res(p64, p64, h, *w[6:12], True)
        return _conv3(p64, h, w[12], w[13], False)

    for o, f in zip(oreg, (f32l, f16, f8, f4)):
        o[...] = head(f, regw)
    for o, f in zip(ocls, (f32l, f16, f8, f4)):
        o[...] = head(f, clsw)


@functools.lru_cache(maxsize=None)
def _get_tail_call(B, bb):
    shapes = ((8, 8), (4, 4), (2, 2), (1, 1))
    out_shape = tuple(jax.ShapeDtypeStruct((B, h, w, 24), jnp.float32)
                      for h, w in shapes)
    out_shape += tuple(jax.ShapeDtypeStruct((B, h, w, 126), jnp.float32)
                       for h, w in shapes)
    out_specs = tuple(_bspec(bb, (h, w, 24)) for h, w in shapes)
    out_specs += tuple(_bspec(bb, (h, w, 126)) for h, w in shapes)
    return lambda *a: pl.pallas_call(
        _tail_body,
        out_shape=out_shape,
        grid=(B // bb,),
        in_specs=[_bspec(bb, (16, 16, 64))] + [_full(x.shape) for x in a[1:]],
        out_specs=out_specs,
        scratch_shapes=[pltpu.VMEM((bb, 10, 8 + _PW, 192), _BF),
                        pltpu.VMEM((bb, 4, 2 + _PW, 384), _BF)],
        compiler_params=_cp())(*a)


def kernel(x, pre1_w, pre1_b, pre2_w, pre2_b, rb0_pre_w, rb0_pre_b, rb0_c0_w, rb0_c0_b, rb0_c1_w, rb0_c1_b, rb1_pre_w, rb1_pre_b, rb1_c0_w, rb1_c0_b, rb1_c1_w, rb1_c1_b, rb2_pre_w, rb2_pre_b, rb2_c0_w, rb2_c0_b, rb2_c1_w, rb2_c1_b, rb3_pre_w, rb3_pre_b, rb3_c0_w, rb3_c0_b, rb3_c1_w, rb3_c1_b, rb4_pre_w, rb4_pre_b, rb4_c0_w, rb4_c0_b, rb4_c1_w, rb4_c1_b, rb5_pre_w, rb5_pre_b, rb5_c0_w, rb5_c0_b, rb5_c1_w, rb5_c1_b, rb6_pre_w, rb6_pre_b, rb6_c0_w, rb6_c0_b, rb6_c1_w, rb6_c1_b, fpn_skip32_w, fpn_skip32_b, fpn_skip16_w, fpn_skip16_b, fpn_skip8_w, fpn_skip8_b, fpn_skip4_w, fpn_skip4_b, fpn_up4_w, fpn_up4_b, fpn_proc8_w, fpn_proc8_b, fpn_up8_w, fpn_up8_b, fpn_proc16_w, fpn_proc16_b, fpn_up16_w, fpn_up16_b, fpn_proc32_w, fpn_proc32_b, cls_res1_pre_w, cls_res1_pre_b, cls_res1_c0_w, cls_res1_c0_b, cls_res1_c1_w, cls_res1_c1_b, cls_res2_pre_w, cls_res2_pre_b, cls_res2_c0_w, cls_res2_c0_b, cls_res2_c1_w, cls_res2_c1_b, cls_conv5_w, cls_conv5_b, reg_res1_pre_w, reg_res1_pre_b, reg_res1_c0_w, reg_res1_c0_b, reg_res1_c1_w, reg_res1_c1_b, reg_res2_pre_w, reg_res2_pre_b, reg_res2_c0_w, reg_res2_c0_b, reg_res2_c1_w, reg_res2_c1_b, reg_conv5_w, reg_conv5_b):
    B = x.shape[0]
    bb = _BB if B % _BB == 0 else 1

    xh = jnp.transpose(x, (0, 2, 3, 1)).astype(jnp.float32)
    xp = jnp.pad(xh, ((0, 0), (1, 1), (1, 1), (0, 0)))
    cols = jnp.concatenate(
        [xp[:, dy:dy + 64, dx:dx + 64, :] for dy in range(3) for dx in range(3)],
        axis=-1).astype(_BF)

    def wb(w, b):
        return [w.astype(_BF), b.reshape(1, -1)]

    def wb3(w, b):
        c3 = w.shape[0] * w.shape[2]
        ws = w.transpose(1, 0, 2, 3).reshape(3, c3, w.shape[3]).astype(_BF)
        return [ws, b.reshape(1, -1)]

    stem_args = [cols, pre1_w.reshape(27, 64).astype(_BF), pre1_b.reshape(1, -1)]
    for w, b in ((pre2_w, pre2_b),
                 (rb0_pre_w, rb0_pre_b), (rb0_c0_w, rb0_c0_b), (rb0_c1_w, rb0_c1_b),
                 (rb1_pre_w, rb1_pre_b), (rb1_c0_w, rb1_c0_b), (rb1_c1_w, rb1_c1_b),
                 (rb2_pre_w, rb2_pre_b), (rb2_c0_w, rb2_c0_b), (rb2_c1_w, rb2_c1_b)):
        stem_args += wb3(w, b)
    enc = _get_stem_call(B)(*stem_args)

    tail_args = [enc]
    for kind, w, b in (
            (3, rb3_pre_w, rb3_pre_b), (3, rb3_c0_w, rb3_c0_b), (3, rb3_c1_w, rb3_c1_b),
            (3, rb4_pre_w, rb4_pre_b), (3, rb4_c0_w, rb4_c0_b), (3, rb4_c1_w, rb4_c1_b),
            (3, rb5_pre_w, rb5_pre_b), (3, rb5_c0_w, rb5_c0_b), (3, rb5_c1_w, rb5_c1_b),
            (3, rb6_pre_w, rb6_pre_b), (3, rb6_c0_w, rb6_c0_b), (3, rb6_c1_w, rb6_c1_b),
            (1, fpn_skip32_w, fpn_skip32_b), (1, fpn_skip16_w, fpn_skip16_b),
            (1, fpn_skip8_w, fpn_skip8_b), (1, fpn_skip4_w, fpn_skip4_b),
            (1, fpn_up4_w, fpn_up4_b), (3, fpn_proc8_w, fpn_proc8_b),
            (1, fpn_up8_w, fpn_up8_b), (3, fpn_proc16_w, fpn_proc16_b),
            (1, fpn_up16_w, fpn_up16_b), (3, fpn_proc32_w, fpn_proc32_b),
            (3, cls_res1_pre_w, cls_res1_pre_b), (3, cls_res1_c0_w, cls_res1_c0_b),
            (3, cls_res1_c1_w, cls_res1_c1_b),
            (3, cls_res2_pre_w, cls_res2_pre_b), (3, cls_res2_c0_w, cls_res2_c0_b),
            (3, cls_res2_c1_w, cls_res2_c1_b),
            (3, cls_conv5_w, cls_conv5_b),
            (3, reg_res1_pre_w, reg_res1_pre_b), (3, reg_res1_c0_w, reg_res1_c0_b),
            (3, reg_res1_c1_w, reg_res1_c1_b),
            (3, reg_res2_pre_w, reg_res2_pre_b), (3, reg_res2_c0_w, reg_res2_c0_b),
            (3, reg_res2_c1_w, reg_res2_c1_b),
            (3, reg_conv5_w, reg_conv5_b)):
        tail_args += wb3(w, b) if kind == 3 else wb(w, b)
    outs = _get_tail_call(B, bb)(*tail_args)
    reg_outs, cls_outs = outs[:4], outs[4:]

    def flat(o, k):
        Bo, H, W, C = o.shape
        return o.reshape(Bo, H * W * (C // k), k)

    regression = jnp.concatenate([flat(o, 4) for o in reg_outs], axis=1)
    classification = jnp.concatenate([flat(o, 21) for o in cls_outs], axis=1)
    return regression, classification
